# baseline (speedup 1.0000x reference)
import jax
import jax.numpy as jnp
from jax.experimental import pallas as pl
from jax.experimental.pallas import tpu as pltpu

L_IN = 10178
PH = 120
M_PAD = 91
U1 = 24
W4 = 81
G = 4
_POOL_W = ((1, 88), (2, 87), (4, 86), (8, 85), (16, 84), (32, 82))


def _ae_kernel(x_ref, w1_ref, b1_ref, wc2_ref, b2_ref,
               wf0_ref, wf1_ref, bf_ref, s0_ref, s1_ref,
               z_ref, x44_ref):
    for i in range(G):
        xr = x_ref[i]
        x0, x1 = xr[0], xr[1]
        xs = jnp.concatenate([x0[0:89], x0[1:90], x0[2:91],
                              x1[0:89], x1[1:90], x1[2:91]], axis=1)
        m = jnp.dot(xs, w1_ref[...], preferred_element_type=jnp.float32) + b1_ref[...]

        for sht, w in _POOL_W:
            a0, r = sht // U1, sht % U1
            sh = jnp.concatenate([m[a0:a0 + w, 16 * r:U1 * 16],
                                  m[a0 + 1:a0 + 1 + w, 0:16 * r]], axis=1)
            m = jnp.maximum(m[0:w, :], sh)
        p1 = jnp.concatenate([m[:, 0:16], m[:, 48:64],
                              m[:, 192:208], m[:, 240:256]], axis=1)
        p1 = jnp.maximum(p1, 0.0).astype(jnp.bfloat16)

        c2 = jnp.dot(p1, wc2_ref[...], preferred_element_type=jnp.float32) + b2_ref[...]
        z = jnp.maximum(jnp.maximum(c2[0:W4, 0:32], c2[0:W4, 32:64]),
                        c2[1:W4 + 1, 0:32])
        z = jnp.maximum(z, 0.0)
        zb = z.astype(jnp.bfloat16)
        z_ref[i] = zb

        t0 = jnp.dot(s0_ref[...], zb, preferred_element_type=jnp.float32)
        t1 = jnp.dot(s1_ref[...], zb, preferred_element_type=jnp.float32)
        slab = (jnp.dot(t0.astype(jnp.bfloat16), wf0_ref[...],
                        preferred_element_type=jnp.float32)
                + jnp.dot(t1.astype(jnp.bfloat16), wf1_ref[...],
                          preferred_element_type=jnp.float32)
                + bf_ref[...])
        x44_ref[i] = jnp.maximum(slab, 0.0).astype(jnp.bfloat16)


def _head_kernel(zf_ref, x44f_ref, wcon_ref, bcon_ref, wfl_ref, bfl_ref,
                 wout_ref, bout_ref, out_ref, x6_ref):
    B = x44f_ref.shape[0]
    x5 = jnp.dot(zf_ref[...], wcon_ref[...],
                 preferred_element_type=jnp.float32) + bcon_ref[...]
    x5 = jnp.maximum(x5, 0.0)
    x54 = jnp.dot(x44f_ref[...], wfl_ref[...],
                  preferred_element_type=jnp.float32) + bfl_ref[...]
    x54 = jnp.maximum(x54, 0.0)
    x6 = jnp.concatenate([x5[0:B], x5[B:2 * B], x5[2 * B:3 * B], x54], axis=1)
    x6_ref[...] = x6
    out_ref[...] = jnp.dot(x6.astype(jnp.bfloat16), wout_ref[...],
                           preferred_element_type=jnp.float32) + bout_ref[...]


def kernel(x, w1q, b1s, wc2, b2s, wf0, wf1, bf, s0, s1,
           wcon, bcon, wfl, bfl, wout, bout):
    B, C, S, L = x.shape
    NS = 3 * B
    BG = B // G
    bf16 = jnp.bfloat16

    xb = jnp.pad(x[:, 0].astype(bf16), ((0, 0), (0, 0), (0, M_PAD * PH - L)))
    xm = xb.reshape(B, S, M_PAD, PH)

    w1 = jnp.concatenate(
        [w1q[q][:, c * PH:(c + 1) * PH] for c in range(2) for q in range(3)],
        axis=1).T.astype(bf16)
    b1t = b1s.reshape(1, U1 * 16)
    wc2t, b2t = wc2.T.astype(bf16), b2s.reshape(1, 64)
    wf0t, wf1t = wf0.T.astype(bf16), wf1.T.astype(bf16)
    s0t, s1t = s0.T.astype(bf16), s1.T.astype(bf16)
    bft = bf.reshape(1, 32)
    nc = wout.shape[1]

    zt, x44t = pl.pallas_call(
        _ae_kernel,
        out_shape=(jax.ShapeDtypeStruct((NS, W4, 32), bf16),
                   jax.ShapeDtypeStruct((NS, 27, 32), bf16)),
        grid=(NS // G,),
        in_specs=[
            pl.BlockSpec((G, 2, M_PAD, PH), lambda g, BG=BG: (g % BG, g // BG, 0, 0)),
            pl.BlockSpec((3 * 2 * PH, U1 * 16), lambda g: (0, 0)),
            pl.BlockSpec((1, U1 * 16), lambda g: (0, 0)),
            pl.BlockSpec((64, 64), lambda g: (0, 0)),
            pl.BlockSpec((1, 64), lambda g: (0, 0)),
            pl.BlockSpec((32, 32), lambda g: (0, 0)),
            pl.BlockSpec((32, 32), lambda g: (0, 0)),
            pl.BlockSpec((1, 32), lambda g: (0, 0)),
            pl.BlockSpec((27, W4), lambda g: (0, 0)),
            pl.BlockSpec((27, W4), lambda g: (0, 0)),
        ],
        out_specs=(pl.BlockSpec((G, W4, 32), lambda g: (g, 0, 0)),
                   pl.BlockSpec((G, 27, 32), lambda g: (g, 0, 0))),
        compiler_params=pltpu.CompilerParams(dimension_semantics=("parallel",)),
    )(xm, w1, b1t, wc2t, b2t, wf0t, wf1t, bft, s0t, s1t)

    z_flat = zt.reshape(NS, W4 * 32)
    x44_flat = x44t.reshape(3, B, 27 * 32).transpose(1, 0, 2).reshape(B, W4 * 32)
    wcon_p = wcon.reshape(32, W4, 32).transpose(1, 0, 2).reshape(W4 * 32, 32)
    wfl_p = wfl.reshape(32, 3, 27, 32).transpose(1, 2, 0, 3).reshape(W4 * 32, 32)

    out, x6 = pl.pallas_call(
        _head_kernel,
        out_shape=(jax.ShapeDtypeStruct((B, nc), jnp.float32),
                   jax.ShapeDtypeStruct((B, 128), jnp.float32)),
        grid=(1,),
        in_specs=[
            pl.BlockSpec((NS, W4 * 32), lambda i: (0, 0)),
            pl.BlockSpec((B, W4 * 32), lambda i: (0, 0)),
            pl.BlockSpec((W4 * 32, 32), lambda i: (0, 0)),
            pl.BlockSpec((1, 32), lambda i: (0, 0)),
            pl.BlockSpec((W4 * 32, 32), lambda i: (0, 0)),
            pl.BlockSpec((1, 32), lambda i: (0, 0)),
            pl.BlockSpec((128, nc), lambda i: (0, 0)),
            pl.BlockSpec((1, nc), lambda i: (0, 0)),
        ],
        out_specs=(pl.BlockSpec((B, nc), lambda i: (0, 0)),
                   pl.BlockSpec((B, 128), lambda i: (0, 0))),
    )(z_flat, x44_flat, wcon_p.astype(bf16), bcon, wfl_p.astype(bf16), bfl,
      wout.astype(bf16), bout)
    return out, x6
